# baseline (speedup 1.0000x reference)
import functools

import jax
import jax.numpy as jnp
from jax import lax
from jax.experimental import pallas as pl
from jax.experimental.pallas import tpu as pltpu
from jax.experimental.pallas import tpu_sc as plsc

_NW = 32
_FW = 16
_UNROLL = 8


def _rsqrt(v):
    i = lax.bitcast_convert_type(v, jnp.int32)
    y = lax.bitcast_convert_type(jnp.int32(0x5F3759DF) - (i >> 1), jnp.float32)
    for _ in range(3):
        y = y * (1.5 - 0.5 * v * y * y)
    return y


def kernel(tensor, weight, bias, batch_num_nodes):
    n, d = tensor.shape
    b = batch_num_nodes.shape[0]
    rpg = n // b
    nfb = d // _FW
    n_tasks = b * nfb
    tasks_per_w = -(-n_tasks // _NW)

    w2 = weight.reshape(nfb, _FW)
    b2 = bias.reshape(nfb, _FW)

    mesh = plsc.VectorSubcoreMesh(core_axis_name="c", subcore_axis_name="s")

    @functools.partial(
        pl.kernel,
        mesh=mesh,
        out_type=jax.ShapeDtypeStruct((n, d), jnp.float32),
        compiler_params=pltpu.CompilerParams(use_tc_tiling_on_sc=False),
        scratch_types=[
            pltpu.VMEM((rpg, _FW), jnp.float32),
            pltpu.VMEM((rpg, _FW), jnp.float32),
            pltpu.VMEM((nfb, _FW), jnp.float32),
            pltpu.VMEM((nfb, _FW), jnp.float32),
            pltpu.SemaphoreType.DMA,
            pltpu.SemaphoreType.DMA,
            pltpu.SemaphoreType.DMA,
            pltpu.SemaphoreType.DMA,
        ],
    )
    def sc_norm(x_hbm, w_hbm, bias_hbm, out_hbm, buf0, buf1, wv, bv,
                isem0, isem1, osem0, osem1):
        wid = lax.axis_index("s") * 2 + lax.axis_index("c")
        pltpu.sync_copy(w_hbm, wv)
        pltpu.sync_copy(bias_hbm, bv)

        bufs = (buf0, buf1)
        isems = (isem0, isem1)
        osems = (osem0, osem1)

        def src(i):
            t = i * _NW + wid
            return x_hbm.at[pl.ds((t >> 3) * rpg, rpg),
                            pl.ds((t & (nfb - 1)) * _FW, _FW)]

        def dst(i):
            t = i * _NW + wid
            return out_hbm.at[pl.ds((t >> 3) * rpg, rpg),
                              pl.ds((t & (nfb - 1)) * _FW, _FW)]

        def guard(i):
            return i * _NW + _NW - 1 < n_tasks

        def maybe(i, fn):
            if guard(i):
                fn()
            else:
                pl.when(i * _NW + wid < n_tasks)(fn)

        def compute(i):
            bi = i % 2
            buf = bufs[bi]
            t = i * _NW + wid
            fb = t & (nfb - 1)
            pltpu.make_async_copy(src(i), buf, isems[bi]).wait()

            zero = jnp.zeros((_FW,), jnp.float32)

            @plsc.parallel_loop(0, rpg, step=_UNROLL, unroll=4,
                                carry=(zero,) * (2 * _UNROLL))
            def acc(base, carry):
                out = []
                for u in range(_UNROLL):
                    x = buf[base + u]
                    out.append(carry[2 * u] + x)
                    out.append(carry[2 * u + 1] + x * x)
                return tuple(out)
            s, q = acc[0], acc[1]
            for u in range(1, _UNROLL):
                s = s + acc[2 * u]
                q = q + acc[2 * u + 1]

            inv_n = jnp.float32(1.0 / rpg)
            mean = s * inv_n
            var = q * inv_n - mean * mean
            rstd = _rsqrt(var + jnp.float32(1e-6))
            scale = wv[fb] * rstd
            shift = bv[fb] - mean * scale

            @plsc.parallel_loop(0, rpg, step=_UNROLL, unroll=4)
            def _norm(base):
                for u in range(_UNROLL):
                    buf[base + u] = buf[base + u] * scale + shift
            pltpu.async_copy(buf, dst(i), osems[bi])

        def start_in(k):
            pltpu.async_copy(src(k), bufs[k % 2], isems[k % 2])

        def wait_out(k):
            pltpu.make_async_copy(bufs[k % 2], dst(k), osems[k % 2]).wait()

        maybe(0, functools.partial(start_in, 0))

        for i in range(tasks_per_w):
            nxt = i + 1
            if nxt < tasks_per_w:
                if nxt - 2 >= 0:
                    maybe(nxt - 2, functools.partial(wait_out, nxt - 2))
                maybe(nxt, functools.partial(start_in, nxt))
            maybe(i, functools.partial(compute, i))

        for i in range(max(0, tasks_per_w - 2), tasks_per_w):
            maybe(i, functools.partial(wait_out, i))

    return sc_norm(tensor, w2, b2)


# speedup vs baseline: 1.0545x; 1.0545x over previous
import functools

import jax
import jax.numpy as jnp
from jax import lax
from jax.experimental import pallas as pl
from jax.experimental.pallas import tpu as pltpu
from jax.experimental.pallas import tpu_sc as plsc

_NW = 32
_FW = 16
_UNROLL = 8


def _rsqrt(v):
    i = lax.bitcast_convert_type(v, jnp.int32)
    y = lax.bitcast_convert_type(jnp.int32(0x5F3759DF) - (i >> 1), jnp.float32)
    for _ in range(3):
        y = y * (1.5 - 0.5 * v * y * y)
    return y


def kernel(tensor, weight, bias, batch_num_nodes):
    n, d = tensor.shape
    b = batch_num_nodes.shape[0]
    rpg = n // b
    nfb = d // _FW
    n_tasks = b * nfb
    tasks_per_w = -(-n_tasks // _NW)

    w2 = weight.reshape(nfb, _FW)
    b2 = bias.reshape(nfb, _FW)

    mesh = plsc.VectorSubcoreMesh(core_axis_name="c", subcore_axis_name="s")

    @functools.partial(
        pl.kernel,
        mesh=mesh,
        out_type=jax.ShapeDtypeStruct((n, d), jnp.float32),
        compiler_params=pltpu.CompilerParams(use_tc_tiling_on_sc=False),
        scratch_types=[
            pltpu.VMEM((rpg, _FW), jnp.float32),
            pltpu.VMEM((rpg, _FW), jnp.float32),
            pltpu.VMEM((rpg, _FW), jnp.float32),
            pltpu.VMEM((nfb, _FW), jnp.float32),
            pltpu.VMEM((nfb, _FW), jnp.float32),
            pltpu.SemaphoreType.DMA,
            pltpu.SemaphoreType.DMA,
            pltpu.SemaphoreType.DMA,
            pltpu.SemaphoreType.DMA,
            pltpu.SemaphoreType.DMA,
            pltpu.SemaphoreType.DMA,
        ],
    )
    def sc_norm(x_hbm, w_hbm, bias_hbm, out_hbm, buf0, buf1, buf2, wv, bv,
                isem0, isem1, isem2, osem0, osem1, osem2):
        wid = lax.axis_index("s") * 2 + lax.axis_index("c")
        pltpu.sync_copy(w_hbm, wv)
        pltpu.sync_copy(bias_hbm, bv)

        bufs = (buf0, buf1, buf2)
        isems = (isem0, isem1, isem2)
        osems = (osem0, osem1, osem2)
        nbuf = len(bufs)

        def src(i):
            t = i * _NW + wid
            return x_hbm.at[pl.ds((t >> 3) * rpg, rpg),
                            pl.ds((t & (nfb - 1)) * _FW, _FW)]

        def dst(i):
            t = i * _NW + wid
            return out_hbm.at[pl.ds((t >> 3) * rpg, rpg),
                              pl.ds((t & (nfb - 1)) * _FW, _FW)]

        def guard(i):
            return i * _NW + _NW - 1 < n_tasks

        def maybe(i, fn):
            if guard(i):
                fn()
            else:
                pl.when(i * _NW + wid < n_tasks)(fn)

        def compute(i):
            bi = i % nbuf
            buf = bufs[bi]
            t = i * _NW + wid
            fb = t & (nfb - 1)
            pltpu.make_async_copy(src(i), buf, isems[bi]).wait()

            zero = jnp.zeros((_FW,), jnp.float32)

            @plsc.parallel_loop(0, rpg, step=_UNROLL, unroll=2,
                                carry=(zero,) * (2 * _UNROLL))
            def acc(base, carry):
                out = []
                for u in range(_UNROLL):
                    x = buf[base + u]
                    out.append(carry[2 * u] + x)
                    out.append(carry[2 * u + 1] + x * x)
                return tuple(out)
            s, q = acc[0], acc[1]
            for u in range(1, _UNROLL):
                s = s + acc[2 * u]
                q = q + acc[2 * u + 1]

            inv_n = jnp.float32(1.0 / rpg)
            mean = s * inv_n
            var = q * inv_n - mean * mean
            rstd = _rsqrt(var + jnp.float32(1e-6))
            scale = wv[fb] * rstd
            shift = bv[fb] - mean * scale

            @plsc.parallel_loop(0, rpg, step=_UNROLL, unroll=2)
            def _norm(base):
                for u in range(_UNROLL):
                    buf[base + u] = buf[base + u] * scale + shift
            pltpu.async_copy(buf, dst(i), osems[bi])

        def start_in(k):
            pltpu.async_copy(src(k), bufs[k % nbuf], isems[k % nbuf])

        def wait_out(k):
            pltpu.make_async_copy(bufs[k % nbuf], dst(k),
                                  osems[k % nbuf]).wait()

        for k in range(min(nbuf - 1, tasks_per_w)):
            maybe(k, functools.partial(start_in, k))

        for i in range(tasks_per_w):
            nxt = i + nbuf - 1
            if nxt < tasks_per_w:
                if nxt - nbuf >= 0:
                    maybe(nxt - nbuf, functools.partial(wait_out, nxt - nbuf))
                maybe(nxt, functools.partial(start_in, nxt))
            maybe(i, functools.partial(compute, i))

        for i in range(max(0, tasks_per_w - nbuf), tasks_per_w):
            maybe(i, functools.partial(wait_out, i))

    return sc_norm(tensor, w2, b2)


# speedup vs baseline: 1.0572x; 1.0026x over previous
import functools

import jax
import jax.numpy as jnp
from jax import lax
from jax.experimental import pallas as pl
from jax.experimental.pallas import tpu as pltpu
from jax.experimental.pallas import tpu_sc as plsc

_NW = 32
_FW = 16
_UNROLL = 8


def _rsqrt(v):
    i = lax.bitcast_convert_type(v, jnp.int32)
    y = lax.bitcast_convert_type(jnp.int32(0x5F3759DF) - (i >> 1), jnp.float32)
    for _ in range(3):
        y = y * (1.5 - 0.5 * v * y * y)
    return y


def kernel(tensor, weight, bias, batch_num_nodes):
    n, d = tensor.shape
    b = batch_num_nodes.shape[0]
    rpg = n // b
    nfb = d // _FW
    n_tasks = b * nfb
    tasks_per_w = -(-n_tasks // _NW)

    w2 = weight.reshape(nfb, _FW)
    b2 = bias.reshape(nfb, _FW)

    mesh = plsc.VectorSubcoreMesh(core_axis_name="c", subcore_axis_name="s")

    @functools.partial(
        pl.kernel,
        mesh=mesh,
        out_type=jax.ShapeDtypeStruct((n, d), jnp.float32),
        compiler_params=pltpu.CompilerParams(use_tc_tiling_on_sc=False),
        scratch_types=[
            pltpu.VMEM((rpg, _FW), jnp.float32),
            pltpu.VMEM((rpg, _FW), jnp.float32),
            pltpu.VMEM((rpg, _FW), jnp.float32),
            pltpu.VMEM((rpg, _FW), jnp.float32),
            pltpu.VMEM((nfb, _FW), jnp.float32),
            pltpu.VMEM((nfb, _FW), jnp.float32),
            pltpu.SemaphoreType.DMA,
            pltpu.SemaphoreType.DMA,
            pltpu.SemaphoreType.DMA,
            pltpu.SemaphoreType.DMA,
            pltpu.SemaphoreType.DMA,
            pltpu.SemaphoreType.DMA,
            pltpu.SemaphoreType.DMA,
            pltpu.SemaphoreType.DMA,
        ],
    )
    def sc_norm(x_hbm, w_hbm, bias_hbm, out_hbm, buf0, buf1, buf2, buf3, wv, bv,
                isem0, isem1, isem2, isem3, osem0, osem1, osem2, osem3):
        wid = lax.axis_index("s") * 2 + lax.axis_index("c")
        pltpu.sync_copy(w_hbm, wv)
        pltpu.sync_copy(bias_hbm, bv)

        bufs = (buf0, buf1, buf2, buf3)
        isems = (isem0, isem1, isem2, isem3)
        osems = (osem0, osem1, osem2, osem3)
        nbuf = len(bufs)

        def src(i):
            t = i * _NW + wid
            return x_hbm.at[pl.ds((t >> 3) * rpg, rpg),
                            pl.ds((t & (nfb - 1)) * _FW, _FW)]

        def dst(i):
            t = i * _NW + wid
            return out_hbm.at[pl.ds((t >> 3) * rpg, rpg),
                              pl.ds((t & (nfb - 1)) * _FW, _FW)]

        def guard(i):
            return i * _NW + _NW - 1 < n_tasks

        def maybe(i, fn):
            if guard(i):
                fn()
            else:
                pl.when(i * _NW + wid < n_tasks)(fn)

        def compute(i):
            bi = i % nbuf
            buf = bufs[bi]
            t = i * _NW + wid
            fb = t & (nfb - 1)
            pltpu.make_async_copy(src(i), buf, isems[bi]).wait()

            zero = jnp.zeros((_FW,), jnp.float32)

            @plsc.parallel_loop(0, rpg, step=_UNROLL, unroll=2,
                                carry=(zero,) * (2 * _UNROLL))
            def acc(base, carry):
                out = []
                for u in range(_UNROLL):
                    x = buf[base + u]
                    out.append(carry[2 * u] + x)
                    out.append(carry[2 * u + 1] + x * x)
                return tuple(out)
            s, q = acc[0], acc[1]
            for u in range(1, _UNROLL):
                s = s + acc[2 * u]
                q = q + acc[2 * u + 1]

            inv_n = jnp.float32(1.0 / rpg)
            mean = s * inv_n
            var = q * inv_n - mean * mean
            rstd = _rsqrt(var + jnp.float32(1e-6))
            scale = wv[fb] * rstd
            shift = bv[fb] - mean * scale

            @plsc.parallel_loop(0, rpg, step=_UNROLL, unroll=2)
            def _norm(base):
                for u in range(_UNROLL):
                    buf[base + u] = buf[base + u] * scale + shift
            pltpu.async_copy(buf, dst(i), osems[bi])

        def start_in(k):
            pltpu.async_copy(src(k), bufs[k % nbuf], isems[k % nbuf])

        def wait_out(k):
            pltpu.make_async_copy(bufs[k % nbuf], dst(k),
                                  osems[k % nbuf]).wait()

        for k in range(min(nbuf - 1, tasks_per_w)):
            maybe(k, functools.partial(start_in, k))

        for i in range(tasks_per_w):
            nxt = i + nbuf - 1
            if nxt < tasks_per_w:
                if nxt - nbuf >= 0:
                    maybe(nxt - nbuf, functools.partial(wait_out, nxt - nbuf))
                maybe(nxt, functools.partial(start_in, nxt))
            maybe(i, functools.partial(compute, i))

        for i in range(max(0, tasks_per_w - nbuf), tasks_per_w):
            maybe(i, functools.partial(wait_out, i))

    return sc_norm(tensor, w2, b2)


# speedup vs baseline: 1.0750x; 1.0168x over previous
import functools

import jax
import jax.numpy as jnp
from jax import lax
from jax.experimental import pallas as pl
from jax.experimental.pallas import tpu as pltpu
from jax.experimental.pallas import tpu_sc as plsc

_NW = 32
_FW = 16
_UNROLL = 8


def _rsqrt(v):
    i = lax.bitcast_convert_type(v, jnp.int32)
    y = lax.bitcast_convert_type(jnp.int32(0x5F3759DF) - (i >> 1), jnp.float32)
    for _ in range(3):
        y = y * (1.5 - 0.5 * v * y * y)
    return y


def kernel(tensor, weight, bias, batch_num_nodes):
    n, d = tensor.shape
    b = batch_num_nodes.shape[0]
    rpg = n // b
    nfb = d // _FW
    n_tasks = b * nfb
    tasks_per_w = -(-n_tasks // _NW)

    w2 = weight.reshape(nfb, _FW)
    b2 = bias.reshape(nfb, _FW)

    mesh = plsc.VectorSubcoreMesh(core_axis_name="c", subcore_axis_name="s")

    @functools.partial(
        pl.kernel,
        mesh=mesh,
        out_type=jax.ShapeDtypeStruct((n, d), jnp.float32),
        compiler_params=pltpu.CompilerParams(use_tc_tiling_on_sc=False),
        scratch_types=[
            pltpu.VMEM((rpg, _FW), jnp.float32),
            pltpu.VMEM((rpg, _FW), jnp.float32),
            pltpu.VMEM((rpg, _FW), jnp.float32),
            pltpu.VMEM((rpg, _FW), jnp.float32),
            pltpu.VMEM((nfb, _FW), jnp.float32),
            pltpu.VMEM((nfb, _FW), jnp.float32),
            pltpu.SemaphoreType.DMA,
            pltpu.SemaphoreType.DMA,
            pltpu.SemaphoreType.DMA,
            pltpu.SemaphoreType.DMA,
            pltpu.SemaphoreType.DMA,
            pltpu.SemaphoreType.DMA,
            pltpu.SemaphoreType.DMA,
            pltpu.SemaphoreType.DMA,
        ],
    )
    def sc_norm(x_hbm, w_hbm, bias_hbm, out_hbm, buf0, buf1, buf2, buf3, wv, bv,
                isem0, isem1, isem2, isem3, osem0, osem1, osem2, osem3):
        wid = lax.axis_index("s") * 2 + lax.axis_index("c")
        bufs = (buf0, buf1, buf2, buf3)
        isems = (isem0, isem1, isem2, isem3)
        osems = (osem0, osem1, osem2, osem3)
        nbuf = len(bufs)

        def src(i):
            t = i * _NW + wid
            return x_hbm.at[pl.ds((t >> 3) * rpg, rpg),
                            pl.ds((t & (nfb - 1)) * _FW, _FW)]

        def dst(i):
            t = i * _NW + wid
            return out_hbm.at[pl.ds((t >> 3) * rpg, rpg),
                              pl.ds((t & (nfb - 1)) * _FW, _FW)]

        def guard(i):
            return i * _NW + _NW - 1 < n_tasks

        def maybe(i, fn):
            if guard(i):
                fn()
            else:
                pl.when(i * _NW + wid < n_tasks)(fn)

        def compute(i):
            bi = i % nbuf
            buf = bufs[bi]
            t = i * _NW + wid
            fb = t & (nfb - 1)
            pltpu.make_async_copy(src(i), buf, isems[bi]).wait()

            zero = jnp.zeros((_FW,), jnp.float32)

            @plsc.parallel_loop(0, rpg, step=_UNROLL, unroll=2,
                                carry=(zero,) * (2 * _UNROLL))
            def acc(base, carry):
                out = []
                for u in range(_UNROLL):
                    x = buf[base + u]
                    out.append(carry[2 * u] + x)
                    out.append(carry[2 * u + 1] + x * x)
                return tuple(out)
            s, q = acc[0], acc[1]
            for u in range(1, _UNROLL):
                s = s + acc[2 * u]
                q = q + acc[2 * u + 1]

            inv_n = jnp.float32(1.0 / rpg)
            mean = s * inv_n
            var = q * inv_n - mean * mean
            rstd = _rsqrt(var + jnp.float32(1e-6))
            scale = wv[fb] * rstd
            shift = bv[fb] - mean * scale

            @plsc.parallel_loop(0, rpg, step=_UNROLL, unroll=2)
            def _norm(base):
                for u in range(_UNROLL):
                    buf[base + u] = buf[base + u] * scale + shift
            pltpu.async_copy(buf, dst(i), osems[bi])

        def start_in(k):
            pltpu.async_copy(src(k), bufs[k % nbuf], isems[k % nbuf])

        def wait_out(k):
            pltpu.make_async_copy(bufs[k % nbuf], dst(k),
                                  osems[k % nbuf]).wait()

        for k in range(min(nbuf - 1, tasks_per_w)):
            maybe(k, functools.partial(start_in, k))
        pltpu.sync_copy(w_hbm, wv)
        pltpu.sync_copy(bias_hbm, bv)

        for i in range(tasks_per_w):
            nxt = i + nbuf - 1
            if nxt < tasks_per_w:
                if nxt - nbuf >= 0:
                    maybe(nxt - nbuf, functools.partial(wait_out, nxt - nbuf))
                maybe(nxt, functools.partial(start_in, nxt))
            maybe(i, functools.partial(compute, i))

        for i in range(max(0, tasks_per_w - nbuf), tasks_per_w):
            maybe(i, functools.partial(wait_out, i))

    return sc_norm(tensor, w2, b2)


# speedup vs baseline: 1.0758x; 1.0008x over previous
import functools

import jax
import jax.numpy as jnp
from jax import lax
from jax.experimental import pallas as pl
from jax.experimental.pallas import tpu as pltpu
from jax.experimental.pallas import tpu_sc as plsc

_NW = 32
_FW = 16
_UNROLL = 8


def _rsqrt(v):
    i = lax.bitcast_convert_type(v, jnp.int32)
    y = lax.bitcast_convert_type(jnp.int32(0x5F3759DF) - (i >> 1), jnp.float32)
    for _ in range(3):
        y = y * (1.5 - 0.5 * v * y * y)
    return y


def kernel(tensor, weight, bias, batch_num_nodes):
    n, d = tensor.shape
    b = batch_num_nodes.shape[0]
    rpg = n // b
    nfb = d // _FW
    n_tasks = b * nfb
    tasks_per_w = -(-n_tasks // _NW)

    w2 = weight.reshape(nfb, _FW)
    b2 = bias.reshape(nfb, _FW)

    mesh = plsc.VectorSubcoreMesh(core_axis_name="c", subcore_axis_name="s")

    @functools.partial(
        pl.kernel,
        mesh=mesh,
        out_type=jax.ShapeDtypeStruct((n, d), jnp.float32),
        compiler_params=pltpu.CompilerParams(use_tc_tiling_on_sc=False, skip_device_barrier=True),
        scratch_types=[
            pltpu.VMEM((rpg, _FW), jnp.float32),
            pltpu.VMEM((rpg, _FW), jnp.float32),
            pltpu.VMEM((rpg, _FW), jnp.float32),
            pltpu.VMEM((rpg, _FW), jnp.float32),
            pltpu.VMEM((nfb, _FW), jnp.float32),
            pltpu.VMEM((nfb, _FW), jnp.float32),
            pltpu.SemaphoreType.DMA,
            pltpu.SemaphoreType.DMA,
            pltpu.SemaphoreType.DMA,
            pltpu.SemaphoreType.DMA,
            pltpu.SemaphoreType.DMA,
            pltpu.SemaphoreType.DMA,
            pltpu.SemaphoreType.DMA,
            pltpu.SemaphoreType.DMA,
        ],
    )
    def sc_norm(x_hbm, w_hbm, bias_hbm, out_hbm, buf0, buf1, buf2, buf3, wv, bv,
                isem0, isem1, isem2, isem3, osem0, osem1, osem2, osem3):
        wid = lax.axis_index("s") * 2 + lax.axis_index("c")
        bufs = (buf0, buf1, buf2, buf3)
        isems = (isem0, isem1, isem2, isem3)
        osems = (osem0, osem1, osem2, osem3)
        nbuf = len(bufs)

        def src(i):
            t = i * _NW + wid
            return x_hbm.at[pl.ds((t >> 3) * rpg, rpg),
                            pl.ds((t & (nfb - 1)) * _FW, _FW)]

        def dst(i):
            t = i * _NW + wid
            return out_hbm.at[pl.ds((t >> 3) * rpg, rpg),
                              pl.ds((t & (nfb - 1)) * _FW, _FW)]

        def guard(i):
            return i * _NW + _NW - 1 < n_tasks

        def maybe(i, fn):
            if guard(i):
                fn()
            else:
                pl.when(i * _NW + wid < n_tasks)(fn)

        def compute(i):
            bi = i % nbuf
            buf = bufs[bi]
            t = i * _NW + wid
            fb = t & (nfb - 1)
            pltpu.make_async_copy(src(i), buf, isems[bi]).wait()

            zero = jnp.zeros((_FW,), jnp.float32)

            @plsc.parallel_loop(0, rpg, step=_UNROLL, unroll=2,
                                carry=(zero,) * (2 * _UNROLL))
            def acc(base, carry):
                out = []
                for u in range(_UNROLL):
                    x = buf[base + u]
                    out.append(carry[2 * u] + x)
                    out.append(carry[2 * u + 1] + x * x)
                return tuple(out)
            s, q = acc[0], acc[1]
            for u in range(1, _UNROLL):
                s = s + acc[2 * u]
                q = q + acc[2 * u + 1]

            inv_n = jnp.float32(1.0 / rpg)
            mean = s * inv_n
            var = q * inv_n - mean * mean
            rstd = _rsqrt(var + jnp.float32(1e-6))
            scale = wv[fb] * rstd
            shift = bv[fb] - mean * scale

            @plsc.parallel_loop(0, rpg, step=_UNROLL, unroll=2)
            def _norm(base):
                for u in range(_UNROLL):
                    buf[base + u] = buf[base + u] * scale + shift
            pltpu.async_copy(buf, dst(i), osems[bi])

        def start_in(k):
            pltpu.async_copy(src(k), bufs[k % nbuf], isems[k % nbuf])

        def wait_out(k):
            pltpu.make_async_copy(bufs[k % nbuf], dst(k),
                                  osems[k % nbuf]).wait()

        for k in range(min(nbuf - 1, tasks_per_w)):
            maybe(k, functools.partial(start_in, k))
        pltpu.sync_copy(w_hbm, wv)
        pltpu.sync_copy(bias_hbm, bv)

        for i in range(tasks_per_w):
            nxt = i + nbuf - 1
            if nxt < tasks_per_w:
                if nxt - nbuf >= 0:
                    maybe(nxt - nbuf, functools.partial(wait_out, nxt - nbuf))
                maybe(nxt, functools.partial(start_in, nxt))
            maybe(i, functools.partial(compute, i))

        for i in range(max(0, tasks_per_w - nbuf), tasks_per_w):
            maybe(i, functools.partial(wait_out, i))

    return sc_norm(tensor, w2, b2)


# speedup vs baseline: 1.1212x; 1.0422x over previous
import functools

import jax
import jax.numpy as jnp
from jax import lax
from jax.experimental import pallas as pl
from jax.experimental.pallas import tpu as pltpu
from jax.experimental.pallas import tpu_sc as plsc

_NW = 32
_FW = 16
_UNROLL = 8


def _rsqrt(v):
    i = lax.bitcast_convert_type(v, jnp.int32)
    y = lax.bitcast_convert_type(jnp.int32(0x5F3759DF) - (i >> 1), jnp.float32)
    for _ in range(2):
        y = y * (1.5 - 0.5 * v * y * y)
    return y


def kernel(tensor, weight, bias, batch_num_nodes):
    n, d = tensor.shape
    b = batch_num_nodes.shape[0]
    rpg = n // b
    nfb = d // _FW
    n_tasks = b * nfb
    tasks_per_w = -(-n_tasks // _NW)

    w2 = weight.reshape(nfb, _FW)
    b2 = bias.reshape(nfb, _FW)

    mesh = plsc.VectorSubcoreMesh(core_axis_name="c", subcore_axis_name="s")

    @functools.partial(
        pl.kernel,
        mesh=mesh,
        out_type=jax.ShapeDtypeStruct((n, d), jnp.float32),
        compiler_params=pltpu.CompilerParams(use_tc_tiling_on_sc=False),
        scratch_types=[
            pltpu.VMEM((rpg, _FW), jnp.float32),
            pltpu.VMEM((rpg, _FW), jnp.float32),
            pltpu.VMEM((rpg, _FW), jnp.float32),
            pltpu.VMEM((rpg, _FW), jnp.float32),
            pltpu.VMEM((nfb, _FW), jnp.float32),
            pltpu.VMEM((nfb, _FW), jnp.float32),
            pltpu.SemaphoreType.DMA,
            pltpu.SemaphoreType.DMA,
            pltpu.SemaphoreType.DMA,
            pltpu.SemaphoreType.DMA,
            pltpu.SemaphoreType.DMA,
            pltpu.SemaphoreType.DMA,
            pltpu.SemaphoreType.DMA,
            pltpu.SemaphoreType.DMA,
        ],
    )
    def sc_norm(x_hbm, w_hbm, bias_hbm, out_hbm, buf0, buf1, buf2, buf3, wv, bv,
                isem0, isem1, isem2, isem3, osem0, osem1, osem2, osem3):
        wid = lax.axis_index("s") * 2 + lax.axis_index("c")
        bufs = (buf0, buf1, buf2, buf3)
        isems = (isem0, isem1, isem2, isem3)
        osems = (osem0, osem1, osem2, osem3)
        nbuf = len(bufs)

        def src(i):
            t = i * _NW + wid
            return x_hbm.at[pl.ds((t >> 3) * rpg, rpg),
                            pl.ds((t & (nfb - 1)) * _FW, _FW)]

        def dst(i):
            t = i * _NW + wid
            return out_hbm.at[pl.ds((t >> 3) * rpg, rpg),
                              pl.ds((t & (nfb - 1)) * _FW, _FW)]

        def guard(i):
            return i * _NW + _NW - 1 < n_tasks

        def maybe(i, fn):
            if guard(i):
                fn()
            else:
                pl.when(i * _NW + wid < n_tasks)(fn)

        def compute(i):
            bi = i % nbuf
            buf = bufs[bi]
            t = i * _NW + wid
            fb = t & (nfb - 1)
            pltpu.make_async_copy(src(i), buf, isems[bi]).wait()

            zero = jnp.zeros((_FW,), jnp.float32)

            @plsc.parallel_loop(0, rpg, step=_UNROLL, unroll=2,
                                carry=(zero,) * (2 * _UNROLL))
            def acc(base, carry):
                out = []
                for u in range(_UNROLL):
                    x = buf[base + u]
                    out.append(carry[2 * u] + x)
                    out.append(carry[2 * u + 1] + x * x)
                return tuple(out)
            sums = [acc[2 * u] for u in range(_UNROLL)]
            sqs = [acc[2 * u + 1] for u in range(_UNROLL)]
            while len(sums) > 1:
                sums = [a + c for a, c in zip(sums[::2], sums[1::2])]
                sqs = [a + c for a, c in zip(sqs[::2], sqs[1::2])]
            s, q = sums[0], sqs[0]

            inv_n = jnp.float32(1.0 / rpg)
            mean = s * inv_n
            var = q * inv_n - mean * mean
            rstd = _rsqrt(var + jnp.float32(1e-6))
            scale = wv[fb] * rstd
            shift = bv[fb] - mean * scale

            half = rpg // 2
            t0 = (t >> 3) * rpg
            c0 = fb * _FW
            for h in range(2):
                @plsc.parallel_loop(h * half, (h + 1) * half, step=_UNROLL,
                                    unroll=2)
                def _norm(base):
                    for u in range(_UNROLL):
                        buf[base + u] = buf[base + u] * scale + shift
                pltpu.async_copy(
                    buf.at[pl.ds(h * half, half)],
                    out_hbm.at[pl.ds(t0 + h * half, half), pl.ds(c0, _FW)],
                    osems[bi])

        def start_in(k):
            pltpu.async_copy(src(k), bufs[k % nbuf], isems[k % nbuf])

        def wait_out(k):
            pltpu.make_async_copy(bufs[k % nbuf], dst(k),
                                  osems[k % nbuf]).wait()

        for k in range(min(nbuf - 1, tasks_per_w)):
            maybe(k, functools.partial(start_in, k))
        pltpu.sync_copy(w_hbm, wv)
        pltpu.sync_copy(bias_hbm, bv)

        for i in range(tasks_per_w):
            nxt = i + nbuf - 1
            if nxt < tasks_per_w:
                if nxt - nbuf >= 0:
                    maybe(nxt - nbuf, functools.partial(wait_out, nxt - nbuf))
                maybe(nxt, functools.partial(start_in, nxt))
            maybe(i, functools.partial(compute, i))

        for i in range(max(0, tasks_per_w - nbuf), tasks_per_w):
            maybe(i, functools.partial(wait_out, i))

    return sc_norm(tensor, w2, b2)
